# baseline (speedup 1.0000x reference)
import math

import jax
import jax.numpy as jnp
from jax.experimental import pallas as pl
from jax.experimental.pallas import tpu as pltpu

B = 64
D_IN = 1024
HID = 512
U = 8
K = 4
IKD = 64
IVD = 512
IH = 4
CKD = 64
CH = 4


def _bf(a):
    return a.astype(jnp.bfloat16)


def _dot(a, b):
    return jnp.dot(_bf(a), _bf(b), preferred_element_type=jnp.float32)


def _stage1_kernel(x_ref, hs_ref, Wk_ref, Wv_ref, Wq_ref,
                   v_ref, a_ref, m_ref):
    x = x_ref[...]
    kx = _dot(x, Wk_ref[...])
    Wv = Wv_ref[...]
    Wvm = (Wv[:, :IVD] + Wv[:, IVD:2 * IVD]
           + Wv[:, 2 * IVD:3 * IVD] + Wv[:, 3 * IVD:]) * 0.25
    v_ref[...] = _dot(x, Wvm)
    kxb = _bf(kx).astype(jnp.float32)
    s_cols = []
    for u in range(U):
        q_u = _dot(hs_ref[:, u * HID:(u + 1) * HID], Wq_ref[u])
        qb = _bf(q_u).astype(jnp.float32)
        s_cols.append(jnp.sum(qb * kxb, axis=1, keepdims=True) * (1.0 / 32.0))
    s = jnp.concatenate(s_cols, axis=1)

    gt = s[:, None, :] > s[:, :, None]
    eq = s[:, None, :] == s[:, :, None]
    j_idx = jax.lax.broadcasted_iota(jnp.int32, (B, U, U), 2)
    i_idx = jax.lax.broadcasted_iota(jnp.int32, (B, U, U), 1)
    before = jnp.logical_or(gt, jnp.logical_and(eq, j_idx < i_idx))
    rank = jnp.sum(before.astype(jnp.float32), axis=2)
    mask = (rank < float(K)).astype(jnp.float32)

    a = jax.nn.sigmoid(s) * mask
    for u in range(U):
        a_ref[u] = a[:, u:u + 1]
        m_ref[u] = mask[:, u:u + 1]


def _fused_kernel(hs_ref, cs_ref, v_ref, a_ref, m_ref,
                  i2h_ref, h2h_ref, Ck_ref, Cq_ref, Cv_ref, Co_ref,
                  csout_ref, hsout_ref,
                  key_scr, qry_scr, val_scr, hb_scr):
    i = pl.program_id(0)

    @pl.when(i < U)
    def _phase1():
        u = i
        hs = hs_ref[...]
        cs = cs_ref[...]
        a = a_ref[u]
        m = m_ref[u]
        inp = a * v_ref[...]
        preact = _dot(inp, i2h_ref[0]) + _dot(hs, h2h_ref[0])
        i_t = jax.nn.sigmoid(preact[:, :HID])
        f_t = jax.nn.sigmoid(preact[:, HID:2 * HID])
        o_t = jax.nn.sigmoid(preact[:, 2 * HID:3 * HID])
        g_t = jnp.tanh(preact[:, 3 * HID:])
        c_t = cs * f_t + i_t * g_t
        h_t = o_t * jnp.tanh(c_t)
        csout_ref[...] = m * c_t + (1.0 - m) * cs
        hb_scr[u] = m * h_t + (1.0 - m) * hs
        key_scr[u] = _dot(h_t, Ck_ref[0])
        qry_scr[u] = _dot(h_t, Cq_ref[0])
        val_scr[u] = _dot(h_t, Cv_ref[0])

    @pl.when(i >= U)
    def _phase2():
        u = i - U
        q = qry_scr[u]
        m = m_ref[u]
        inv = 1.0 / math.sqrt(CKD)
        ctx_parts = []
        for ch in range(CH):
            qh = q[:, ch * CKD:(ch + 1) * CKD]
            sc_cols = []
            for up in range(U):
                kh = key_scr[up, :, ch * CKD:(ch + 1) * CKD]
                sc_cols.append(jnp.sum(qh * kh, axis=1, keepdims=True))
            sc = jnp.concatenate(sc_cols, axis=1) * inv
            sc = sc - jnp.max(sc, axis=1, keepdims=True)
            e = jnp.exp(sc)
            p = e / jnp.sum(e, axis=1, keepdims=True)
            ctx_h = p[:, 0:1] * val_scr[0, :, ch * HID:(ch + 1) * HID]
            for up in range(1, U):
                ctx_h = ctx_h + p[:, up:up + 1] * val_scr[up, :, ch * HID:(ch + 1) * HID]
            ctx_parts.append(ctx_h)
        ctx = jnp.concatenate(ctx_parts, axis=1)
        delta = _dot(ctx, Co_ref[0])
        hsout_ref[...] = hb_scr[u] + m * delta


def kernel(x, hs, cs, Wk, Wv, Wq, i2h, h2h, Ck, Cq, Cv, Co):
    x2 = x[:, 0, :]
    hs2 = hs.reshape(B, U * HID)
    cs2 = cs.reshape(B, U * HID)

    v, a_t, m_t = pl.pallas_call(
        _stage1_kernel,
        out_shape=[
            jax.ShapeDtypeStruct((B, IVD), jnp.float32),
            jax.ShapeDtypeStruct((U, B, 1), jnp.float32),
            jax.ShapeDtypeStruct((U, B, 1), jnp.float32),
        ],
    )(x2, hs2, Wk, Wv, Wq)

    lo = lambda i: jnp.minimum(i, U - 1)
    hi = lambda i: jnp.maximum(i - U, 0)
    unit_cols = lambda ph: pl.BlockSpec(
        (B, HID), (lambda i: (0, lo(i))) if ph == 1 else (lambda i: (0, hi(i))))
    wblock = lambda d_in, d_out, ph: pl.BlockSpec(
        (1, d_in, d_out),
        (lambda i: (lo(i), 0, 0)) if ph == 1 else (lambda i: (hi(i), 0, 0)))
    full2 = lambda r, c: pl.BlockSpec((r, c), lambda i: (0, 0))
    full3 = lambda a_, b_, c_: pl.BlockSpec((a_, b_, c_), lambda i: (0, 0, 0))

    csout2, hsout2 = pl.pallas_call(
        _fused_kernel,
        grid=(2 * U,),
        in_specs=[
            unit_cols(1),
            unit_cols(1),
            full2(B, IVD),
            full3(U, B, 1),
            full3(U, B, 1),
            wblock(IVD, 4 * HID, 1),
            wblock(HID, 4 * HID, 1),
            wblock(HID, CH * CKD, 1),
            wblock(HID, CH * CKD, 1),
            wblock(HID, CH * HID, 1),
            wblock(CH * HID, HID, 2),
        ],
        out_specs=[
            unit_cols(1),
            unit_cols(2),
        ],
        out_shape=[
            jax.ShapeDtypeStruct((B, U * HID), jnp.float32),
            jax.ShapeDtypeStruct((B, U * HID), jnp.float32),
        ],
        scratch_shapes=[
            pltpu.VMEM((U, B, CH * CKD), jnp.float32),
            pltpu.VMEM((U, B, CH * CKD), jnp.float32),
            pltpu.VMEM((U, B, CH * HID), jnp.float32),
            pltpu.VMEM((U, B, HID), jnp.float32),
        ],
    )(hs2, cs2, v, a_t, m_t, i2h, h2h, Ck, Cq, Cv, Co)

    return hsout2.reshape(B, U, HID), csout2.reshape(B, U, HID)


# speedup vs baseline: 1.0027x; 1.0027x over previous
import math

import jax
import jax.numpy as jnp
from jax.experimental import pallas as pl
from jax.experimental.pallas import tpu as pltpu

B = 64
D_IN = 1024
HID = 512
U = 8
K = 4
IKD = 64
IVD = 512
IH = 4
CKD = 64
CH = 4


def _bf(a):
    return a.astype(jnp.bfloat16)


def _dot(a, b):
    return jnp.dot(_bf(a), _bf(b), preferred_element_type=jnp.float32)


def _stage1_kernel(x_ref, hs_ref, Wk_ref, Wv_ref, Wq_ref,
                   v_ref, a_ref, m_ref):
    x = x_ref[...]
    kx = _dot(x, Wk_ref[...])
    Wv = Wv_ref[...]
    Wvm = (Wv[:, :IVD] + Wv[:, IVD:2 * IVD]
           + Wv[:, 2 * IVD:3 * IVD] + Wv[:, 3 * IVD:]) * 0.25
    v_ref[...] = _dot(x, Wvm)
    kxb = _bf(kx).astype(jnp.float32)
    s_cols = []
    for u in range(U):
        q_u = _dot(hs_ref[:, u * HID:(u + 1) * HID], Wq_ref[u])
        qb = _bf(q_u).astype(jnp.float32)
        s_cols.append(jnp.sum(qb * kxb, axis=1, keepdims=True) * (1.0 / 32.0))
    s = jnp.concatenate(s_cols, axis=1)

    gt = s[:, None, :] > s[:, :, None]
    eq = s[:, None, :] == s[:, :, None]
    j_idx = jax.lax.broadcasted_iota(jnp.int32, (B, U, U), 2)
    i_idx = jax.lax.broadcasted_iota(jnp.int32, (B, U, U), 1)
    before = jnp.logical_or(gt, jnp.logical_and(eq, j_idx < i_idx))
    rank = jnp.sum(before.astype(jnp.float32), axis=2)
    mask = (rank < float(K)).astype(jnp.float32)

    a = jax.nn.sigmoid(s) * mask
    for u in range(U):
        a_ref[u] = a[:, u:u + 1]
        m_ref[u] = mask[:, u:u + 1]


def _fused_kernel(hs_ref, cs_ref, v_ref, a_ref, m_ref,
                  i2hA_ref, i2hB_ref, h2hA_ref, h2hB_ref,
                  Ck_ref, Cq_ref, CvA_ref, CvB_ref, CoA_ref, CoB_ref,
                  csout_ref, hsout_ref,
                  key_scr, qry_scr, val_scr, hb_scr):
    i = pl.program_id(0)

    @pl.when(i < U)
    def _phase1():
        u = i
        hs = hs_ref[...]
        cs = cs_ref[...]
        a = a_ref[u]
        m = m_ref[u]
        inp = a * v_ref[...]
        pA = _dot(inp, i2hA_ref[0]) + _dot(hs, h2hA_ref[0])
        pB = _dot(inp, i2hB_ref[0]) + _dot(hs, h2hB_ref[0])
        i_t = jax.nn.sigmoid(pA[:, :HID])
        f_t = jax.nn.sigmoid(pA[:, HID:])
        o_t = jax.nn.sigmoid(pB[:, :HID])
        g_t = jnp.tanh(pB[:, HID:])
        c_t = cs * f_t + i_t * g_t
        h_t = o_t * jnp.tanh(c_t)
        csout_ref[...] = m * c_t + (1.0 - m) * cs
        hb_scr[u] = m * h_t + (1.0 - m) * hs
        key_scr[u] = _dot(h_t, Ck_ref[0])
        qry_scr[u] = _dot(h_t, Cq_ref[0])
        val_scr[u, :, :CH * HID // 2] = _dot(h_t, CvA_ref[0])
        val_scr[u, :, CH * HID // 2:] = _dot(h_t, CvB_ref[0])

    @pl.when(i >= U)
    def _phase2():
        u = i - U
        q = qry_scr[u]
        m = m_ref[u]
        inv = 1.0 / math.sqrt(CKD)
        ctx_parts = []
        for ch in range(CH):
            qh = q[:, ch * CKD:(ch + 1) * CKD]
            sc_cols = []
            for up in range(U):
                kh = key_scr[up, :, ch * CKD:(ch + 1) * CKD]
                sc_cols.append(jnp.sum(qh * kh, axis=1, keepdims=True))
            sc = jnp.concatenate(sc_cols, axis=1) * inv
            sc = sc - jnp.max(sc, axis=1, keepdims=True)
            e = jnp.exp(sc)
            p = e / jnp.sum(e, axis=1, keepdims=True)
            ctx_h = p[:, 0:1] * val_scr[0, :, ch * HID:(ch + 1) * HID]
            for up in range(1, U):
                ctx_h = ctx_h + p[:, up:up + 1] * val_scr[up, :, ch * HID:(ch + 1) * HID]
            ctx_parts.append(ctx_h)
        ctx = jnp.concatenate(ctx_parts, axis=1)
        delta = (_dot(ctx[:, :CH * HID // 2], CoA_ref[0])
                 + _dot(ctx[:, CH * HID // 2:], CoB_ref[0]))
        hsout_ref[...] = hb_scr[u] + m * delta


def kernel(x, hs, cs, Wk, Wv, Wq, i2h, h2h, Ck, Cq, Cv, Co):
    x2 = x[:, 0, :]
    hs2 = hs.reshape(B, U * HID)
    cs2 = cs.reshape(B, U * HID)

    v, a_t, m_t = pl.pallas_call(
        _stage1_kernel,
        out_shape=[
            jax.ShapeDtypeStruct((B, IVD), jnp.float32),
            jax.ShapeDtypeStruct((U, B, 1), jnp.float32),
            jax.ShapeDtypeStruct((U, B, 1), jnp.float32),
        ],
    )(x2, hs2, Wk, Wv, Wq)

    lo = lambda i: jnp.minimum(i, U - 1)
    hi = lambda i: jnp.maximum(i - U, 0)
    unit_cols = lambda ph: pl.BlockSpec(
        (B, HID), (lambda i: (0, lo(i))) if ph == 1 else (lambda i: (0, hi(i))))
    whalf = lambda d_in, d_out, half, ph: pl.BlockSpec(
        (1, d_in, d_out // 2),
        (lambda i: (lo(i), 0, half)) if ph == 1 else (lambda i: (hi(i), 0, half)))
    wblock = lambda d_in, d_out, ph: pl.BlockSpec(
        (1, d_in, d_out),
        (lambda i: (lo(i), 0, 0)) if ph == 1 else (lambda i: (hi(i), 0, 0)))
    full2 = lambda r, c: pl.BlockSpec((r, c), lambda i: (0, 0))
    full3 = lambda a_, b_, c_: pl.BlockSpec((a_, b_, c_), lambda i: (0, 0, 0))

    csout2, hsout2 = pl.pallas_call(
        _fused_kernel,
        grid=(2 * U,),
        in_specs=[
            unit_cols(1),
            unit_cols(1),
            full2(B, IVD),
            full3(U, B, 1),
            full3(U, B, 1),
            whalf(IVD, 4 * HID, 0, 1),
            whalf(IVD, 4 * HID, 1, 1),
            whalf(HID, 4 * HID, 0, 1),
            whalf(HID, 4 * HID, 1, 1),
            wblock(HID, CH * CKD, 1),
            wblock(HID, CH * CKD, 1),
            whalf(HID, CH * HID, 0, 1),
            whalf(HID, CH * HID, 1, 1),
            pl.BlockSpec((1, CH * HID // 2, HID),
                         lambda i: (hi(i), 0, 0)),
            pl.BlockSpec((1, CH * HID // 2, HID),
                         lambda i: (hi(i), 1, 0)),
        ],
        out_specs=[
            unit_cols(1),
            unit_cols(2),
        ],
        out_shape=[
            jax.ShapeDtypeStruct((B, U * HID), jnp.float32),
            jax.ShapeDtypeStruct((B, U * HID), jnp.float32),
        ],
        scratch_shapes=[
            pltpu.VMEM((U, B, CH * CKD), jnp.float32),
            pltpu.VMEM((U, B, CH * CKD), jnp.float32),
            pltpu.VMEM((U, B, CH * HID), jnp.float32),
            pltpu.VMEM((U, B, HID), jnp.float32),
        ],
    )(hs2, cs2, v, a_t, m_t, i2h, i2h, h2h, h2h, Ck, Cq, Cv, Cv, Co, Co)

    return hsout2.reshape(B, U, HID), csout2.reshape(B, U, HID)
